# baseline (speedup 1.0000x reference)
import jax
import jax.numpy as jnp
from jax import lax
from jax.experimental import pallas as pl
from jax.experimental.pallas import tpu as pltpu
from jax.experimental.pallas import tpu_sc as plsc

_N = 65536
_C = 1000
_NB = 20
_NW = 32
_RW = _N // _NW
_L = 16
_CH = 16
_NCH = _RW // _CH
_NBUF = 4
_NK = _C // _L


def _scf_body(x_hbm, lab_hbm, bnd_hbm, part_hbm,
              buf0, buf1, buf2, buf3, lab_v, bnd_v, accum,
              sem0, sem1, sem2, sem3):
    c = lax.axis_index("c")
    s = lax.axis_index("s")
    w = s * 2 + c
    rbase = w * _RW

    bufs = [buf0, buf1, buf2, buf3]
    sems = [sem0, sem1, sem2, sem3]

    pltpu.sync_copy(lab_hbm.at[pl.ds(rbase, _RW)], lab_v)
    pltpu.sync_copy(bnd_hbm, bnd_v)

    zeros = jnp.zeros((_L,), jnp.float32)
    ones = jnp.ones((_L,), jnp.float32)
    for b in range(_NB):
        for q in range(3):
            accum[pl.ds((b * 3 + q) * _L, _L)] = zeros

    bv0 = bnd_v[pl.ds(0, _L)]
    bv1 = bnd_v[pl.ds(_L, _L)]
    bs = [bv0[j] for j in range(_L)] + [bv1[j] for j in range(_NB + 1 - _L)]

    def start(buf, sem, ch):
        pltpu.make_async_copy(
            x_hbm.at[pl.ds(rbase + ch * _CH, _CH), :], buf, sem
        ).start()

    def drain(buf, sem):
        pltpu.make_async_copy(
            x_hbm.at[pl.ds(rbase, _CH), :], buf, sem
        ).wait()

    for b in range(_NBUF):
        start(bufs[b], sems[b], b)

    rows = lax.iota(jnp.int32, _L)
    neginf = jnp.full((_L,), -jnp.inf, jnp.float32)
    zi = jnp.zeros((_L,), jnp.int32)
    _A = 8

    def process(buf, ch):
        def jstep(i, carry):
            maxs, cols, colv0 = carry
            maxs = list(maxs)
            cols = list(cols)
            for a in range(_A):
                ca = colv0 + a
                v = plsc.load_gather(buf, [rows, ca])
                m = v > maxs[a]
                maxs[a] = jnp.where(m, v, maxs[a])
                cols[a] = jnp.where(m, ca, cols[a])
            return tuple(maxs), tuple(cols), colv0 + _A

        maxs, cols, _ = lax.fori_loop(
            0, _C // _A, jstep,
            (tuple([neginf] * _A), tuple([zi] * _A), zi))

        confv, colv = maxs[0], cols[0]
        for a in range(1, _A):
            take = (maxs[a] > confv) | ((maxs[a] == confv) & (cols[a] < colv))
            confv = jnp.where(take, maxs[a], confv)
            colv = jnp.where(take, cols[a], colv)

        lab16 = lab_v[pl.ds(ch * _CH, _CH)]
        av = (colv == lab16).astype(jnp.float32)
        cv = confv
        for b in range(_NB):
            m = (cv > bs[b]) & (cv <= bs[b + 1])
            plsc.addupdate(accum.at[pl.ds((b * 3 + 0) * _L, _L)],
                           jnp.where(m, ones, zeros))
            plsc.addupdate(accum.at[pl.ds((b * 3 + 1) * _L, _L)],
                           jnp.where(m, cv, zeros))
            plsc.addupdate(accum.at[pl.ds((b * 3 + 2) * _L, _L)],
                           jnp.where(m, av, zeros))

    def iter_body(it, carry):
        for b in range(_NBUF):
            ch = it * _NBUF + b
            drain(bufs[b], sems[b])
            process(bufs[b], ch)
            nxt = ch + _NBUF

            @pl.when(nxt < _NCH)
            def _():
                start(bufs[b], sems[b], nxt)
        return carry

    lax.fori_loop(0, _NCH // _NBUF, iter_body, 0)

    pltpu.sync_copy(accum, part_hbm.at[pl.ds(w * _NB * 3 * _L, _NB * 3 * _L)])


def _scf_stage(outputs, labels, boundaries):
    mesh = plsc.VectorSubcoreMesh(core_axis_name="c", subcore_axis_name="s")
    return pl.kernel(
        _scf_body,
        out_type=jax.ShapeDtypeStruct((_NW * _NB * 3 * _L,), jnp.float32),
        mesh=mesh,
        compiler_params=pltpu.CompilerParams(needs_layout_passes=False),
        scratch_types=[
            pltpu.VMEM((_CH, _C), jnp.float32),
            pltpu.VMEM((_CH, _C), jnp.float32),
            pltpu.VMEM((_CH, _C), jnp.float32),
            pltpu.VMEM((_CH, _C), jnp.float32),
            pltpu.VMEM((_RW,), jnp.int32),
            pltpu.VMEM((32,), jnp.float32),
            pltpu.VMEM((_NB * 3 * _L,), jnp.float32),
            pltpu.SemaphoreType.DMA,
            pltpu.SemaphoreType.DMA,
            pltpu.SemaphoreType.DMA,
            pltpu.SemaphoreType.DMA,
        ],
    )(outputs, labels, boundaries)


@jax.jit
def kernel(outputs, labels):
    boundaries = jnp.linspace(0.0, 1.0, _NB + 1)
    bnd = jnp.concatenate([boundaries, jnp.full((32 - _NB - 1,), 2.0,
                                                jnp.float32)])
    parts = _scf_stage(outputs, labels, bnd).reshape(_NW, _NB, 3, _L)
    sums = jnp.sum(parts, axis=(0, 3))
    cnt = sums[:, 0]
    conf_s = sums[:, 1]
    acc_s = sums[:, 2]
    safe = jnp.maximum(cnt, 1.0)
    acc_in_bin = jnp.where(cnt > 0, acc_s / safe, 0.0)
    conf_in_bin = jnp.where(cnt > 0, conf_s / safe, 0.0)
    ece = jnp.sum(jnp.abs(conf_in_bin - acc_in_bin) * (cnt / _N))
    return ece.reshape(1)


# speedup vs baseline: 3.1078x; 3.1078x over previous
import jax
import jax.numpy as jnp
from jax import lax
from jax.experimental import pallas as pl
from jax.experimental.pallas import tpu as pltpu
from jax.experimental.pallas import tpu_sc as plsc

_N = 65536
_C = 1000
_NB = 20

_BRM = 512
_NBUF = 6


def _tc_body(x_hbm, lab_ref, conf_ref, acc_ref, x_vmem, sems):
    i = pl.program_id(0)
    nblk = pl.num_programs(0)

    def start(blk, slot):
        pltpu.make_async_copy(
            x_hbm.at[pl.ds(blk * _BRM, _BRM), :],
            x_vmem.at[slot],
            sems.at[slot],
        ).start()

    @pl.when(i == 0)
    def _prologue():
        for k in range(_NBUF - 1):
            start(k, k)

    nxt = i + _NBUF - 1

    @pl.when(nxt < nblk)
    def _issue():
        start(nxt, nxt % _NBUF)

    slot = i % _NBUF
    pltpu.make_async_copy(
        x_hbm.at[pl.ds(i * _BRM, _BRM), :], x_vmem.at[slot], sems.at[slot]
    ).wait()

    x = x_vmem[slot]
    conf = jnp.max(x, axis=1)
    ids = lax.broadcasted_iota(jnp.int32, (_BRM, _C), 1)
    masked = jnp.where(x == conf[:, None], ids, _C)
    pred = jnp.min(masked, axis=1)
    conf_ref[...] = conf
    acc_ref[...] = (pred == lab_ref[...]).astype(jnp.float32)


def _tc_stage(outputs, labels):
    return pl.pallas_call(
        _tc_body,
        grid=(_N // _BRM,),
        in_specs=[
            pl.BlockSpec(memory_space=pl.MemorySpace.ANY),
            pl.BlockSpec((_BRM,), lambda i: (i,)),
        ],
        out_specs=[
            pl.BlockSpec((_BRM,), lambda i: (i,)),
            pl.BlockSpec((_BRM,), lambda i: (i,)),
        ],
        out_shape=[
            jax.ShapeDtypeStruct((_N,), jnp.float32),
            jax.ShapeDtypeStruct((_N,), jnp.float32),
        ],
        scratch_shapes=[
            pltpu.VMEM((_NBUF, _BRM, _C), jnp.float32),
            pltpu.SemaphoreType.DMA((_NBUF,)),
        ],
    )(outputs, labels)


_NW = 32
_PW = _N // _NW
_L = 16


def _sc_body(conf_hbm, acc_hbm, bnd_hbm, part_hbm, conf_v, acc_v, bnd_v,
             accum):
    c = lax.axis_index("c")
    s = lax.axis_index("s")
    w = s * 2 + c
    base = w * _PW

    pltpu.sync_copy(conf_hbm.at[pl.ds(base, _PW)], conf_v)
    pltpu.sync_copy(acc_hbm.at[pl.ds(base, _PW)], acc_v)
    pltpu.sync_copy(bnd_hbm, bnd_v)

    zeros = jnp.zeros((_L,), jnp.float32)
    ones = jnp.ones((_L,), jnp.float32)
    for b in range(_NB):
        for q in range(3):
            accum[0, b, q] = zeros

    bv0 = bnd_v[pl.ds(0, _L)]
    bv1 = bnd_v[pl.ds(_L, _L)]
    bs = [bv0[j] for j in range(_L)] + [bv1[j] for j in range(_NB + 1 - _L)]

    def step(i, carry):
        cv = conf_v[pl.ds(i * _L, _L)]
        av = acc_v[pl.ds(i * _L, _L)]
        for b in range(_NB):
            m = (cv > bs[b]) & (cv <= bs[b + 1])
            plsc.addupdate(accum.at[0, b, 0], jnp.where(m, ones, zeros))
            plsc.addupdate(accum.at[0, b, 1], jnp.where(m, cv, zeros))
            plsc.addupdate(accum.at[0, b, 2], jnp.where(m, av, zeros))
        return carry

    lax.fori_loop(0, _PW // _L, step, 0)

    pltpu.sync_copy(accum, part_hbm.at[pl.ds(w, 1)])


def _sc_stage(conf, acc, boundaries):
    mesh = plsc.VectorSubcoreMesh(core_axis_name="c", subcore_axis_name="s")
    return pl.kernel(
        _sc_body,
        out_type=jax.ShapeDtypeStruct((_NW, _NB, 3, _L), jnp.float32),
        mesh=mesh,
        scratch_types=[
            pltpu.VMEM((_PW,), jnp.float32),
            pltpu.VMEM((_PW,), jnp.float32),
            pltpu.VMEM((32,), jnp.float32),
            pltpu.VMEM((1, _NB, 3, _L), jnp.float32),
        ],
    )(conf, acc, boundaries)


@jax.jit
def kernel(outputs, labels):
    conf, acc = _tc_stage(outputs, labels)
    boundaries = jnp.linspace(0.0, 1.0, _NB + 1)
    bnd = jnp.concatenate([boundaries, jnp.full((32 - _NB - 1,), 2.0,
                                                jnp.float32)])
    parts = _sc_stage(conf, acc, bnd)
    sums = jnp.sum(parts, axis=(0, 3))
    cnt = sums[:, 0]
    conf_s = sums[:, 1]
    acc_s = sums[:, 2]
    safe = jnp.maximum(cnt, 1.0)
    acc_in_bin = jnp.where(cnt > 0, acc_s / safe, 0.0)
    conf_in_bin = jnp.where(cnt > 0, conf_s / safe, 0.0)
    ece = jnp.sum(jnp.abs(conf_in_bin - acc_in_bin) * (cnt / _N))
    return ece.reshape(1)
